# baseline (speedup 1.0000x reference)
import jax
import jax.numpy as jnp
from jax.experimental import pallas as pl
from jax.experimental.pallas import tpu as pltpu


def _gnn_kernel(x_ref, adj_ref, wj_ref, w1_ref, w2_ref, out_ref, acc_ref,
                y_ref):
    k = pl.program_id(1)
    nk = pl.num_programs(1)
    d = w1_ref.shape[0]

    @pl.when(k == 0)
    def _():
        xs0 = x_ref[0, :, pl.ds(0, d)]
        y_ref[...] = jnp.dot(
            xs0, w1_ref[...], preferred_element_type=jnp.float32
        ).astype(jnp.bfloat16)

    y = y_ref[...]

    a = adj_ref[0, 0].astype(jnp.bfloat16)
    m = jnp.dot(a, y, preferred_element_type=jnp.float32)

    @pl.when(k < nk - 1)
    def _():
        xs1 = x_ref[0, :, pl.ds((k + 1) * d, d)]
        y_ref[...] = jnp.dot(
            xs1, w1_ref[...], preferred_element_type=jnp.float32
        ).astype(jnp.bfloat16)

    wj = wj_ref[0, 0]
    phi = jnp.maximum(m, 0.0) * wj

    @pl.when(k == 0)
    def _():
        acc_ref[...] = phi

    @pl.when(k > 0)
    def _():
        acc_ref[...] = acc_ref[...] + phi

    @pl.when(k == nk - 1)
    def _():
        out_ref[0] = jnp.dot(acc_ref[...].astype(jnp.bfloat16), w2_ref[...],
                             preferred_element_type=jnp.float32)


def kernel(x, adj, w_j, W1, W2):
    B, N, K, D = x.shape
    H = W1.shape[1]
    DOUT = W2.shape[1]

    xb = x.astype(jnp.bfloat16).reshape(B, N, K * D)
    w1b = W1.astype(jnp.bfloat16)
    w2b = W2.astype(jnp.bfloat16)
    wjt = jnp.transpose(w_j, (0, 2, 1)).astype(jnp.float32).reshape(B, K, N, 1)

    return pl.pallas_call(
        _gnn_kernel,
        grid=(B, K),
        in_specs=[
            pl.BlockSpec((1, N, K * D), lambda b, k: (b, 0, 0)),
            pl.BlockSpec((1, 1, N, N), lambda b, k: (b, k, 0, 0)),
            pl.BlockSpec((1, 1, N, 1), lambda b, k: (b, k, 0, 0)),
            pl.BlockSpec((D, H), lambda b, k: (0, 0)),
            pl.BlockSpec((H, DOUT), lambda b, k: (0, 0)),
        ],
        out_specs=pl.BlockSpec((1, N, DOUT), lambda b, k: (b, 0, 0)),
        out_shape=jax.ShapeDtypeStruct((B, N, DOUT), jnp.float32),
        scratch_shapes=[pltpu.VMEM((N, H), jnp.float32),
                        pltpu.VMEM((N, H), jnp.bfloat16)],
    )(xb, adj, wjt, w1b, w2b)


# speedup vs baseline: 2.0518x; 2.0518x over previous
import jax
import jax.numpy as jnp
from jax.experimental import pallas as pl
from jax.experimental.pallas import tpu as pltpu


def _gnn_kernel(x_ref, adj_ref, wj_ref, w1_ref, w2_ref, out_ref, acc_ref):
    nk = adj_ref.shape[1]
    wj = wj_ref[0]
    for k in range(nk):
        xs = x_ref[0, :, k, :].astype(jnp.bfloat16)
        y = jnp.dot(xs, w1_ref[...], preferred_element_type=jnp.float32)

        a = adj_ref[0, k].astype(jnp.bfloat16)
        m = jnp.dot(a, y.astype(jnp.bfloat16),
                    preferred_element_type=jnp.float32)

        phi = jnp.maximum(m, 0.0) * wj[:, k:k + 1].astype(jnp.float32)
        if k == 0:
            acc_ref[...] = phi
        else:
            acc_ref[...] = acc_ref[...] + phi

    out_ref[0] = jnp.dot(acc_ref[...].astype(jnp.bfloat16), w2_ref[...],
                         preferred_element_type=jnp.float32)


def kernel(x, adj, w_j, W1, W2):
    B, N, K, D = x.shape
    H = W1.shape[1]
    DOUT = W2.shape[1]

    adj_i8 = adj.view(jnp.int8)
    w1b = W1.astype(jnp.bfloat16)
    w2b = W2.astype(jnp.bfloat16)

    return pl.pallas_call(
        _gnn_kernel,
        grid=(B,),
        in_specs=[
            pl.BlockSpec((1, N, K, D), lambda b: (b, 0, 0, 0)),
            pl.BlockSpec((1, K, N, N), lambda b: (b, 0, 0, 0)),
            pl.BlockSpec((1, N, K), lambda b: (b, 0, 0)),
            pl.BlockSpec((D, H), lambda b: (0, 0)),
            pl.BlockSpec((H, DOUT), lambda b: (0, 0)),
        ],
        out_specs=pl.BlockSpec((1, N, DOUT), lambda b: (b, 0, 0)),
        out_shape=jax.ShapeDtypeStruct((B, N, DOUT), jnp.float32),
        scratch_shapes=[pltpu.VMEM((N, H), jnp.float32)],
    )(x, adj_i8, w_j, w1b, w2b)


# speedup vs baseline: 2.1442x; 1.0451x over previous
import jax
import jax.numpy as jnp
from jax.experimental import pallas as pl
from jax.experimental.pallas import tpu as pltpu


def _gnn_kernel(x_hbm, adj_ref, wj_ref, w1_ref, w2_ref, out_ref,
                acc_ref, xs_ref, sem):
    b = pl.program_id(0)
    nb = pl.num_programs(0)
    nk = adj_ref.shape[1]
    slot = jax.lax.rem(b, 2)
    nxt = jax.lax.rem(b + 1, 2)

    def _issue(bb, ss):
        for k in range(nk):
            pltpu.make_async_copy(
                x_hbm.at[bb, :, k, :], xs_ref.at[ss, k], sem.at[ss, k]
            ).start()

    @pl.when(b == 0)
    def _():
        _issue(0, 0)

    @pl.when(b + 1 < nb)
    def _():
        _issue(b + 1, nxt)

    wj = wj_ref[0]
    for k in range(nk):
        pltpu.make_async_copy(
            x_hbm.at[b, :, k, :], xs_ref.at[slot, k], sem.at[slot, k]
        ).wait()

        xs = xs_ref[slot, k].astype(jnp.bfloat16)
        y = jnp.dot(xs, w1_ref[...], preferred_element_type=jnp.float32)

        a = adj_ref[0, k].astype(jnp.bfloat16)
        m = jnp.dot(a, y.astype(jnp.bfloat16),
                    preferred_element_type=jnp.float32)

        phi = jnp.maximum(m, 0.0) * wj[:, k:k + 1].astype(jnp.float32)
        if k == 0:
            acc_ref[...] = phi
        else:
            acc_ref[...] = acc_ref[...] + phi

    out_ref[0] = jnp.dot(acc_ref[...].astype(jnp.bfloat16), w2_ref[...],
                         preferred_element_type=jnp.float32)


def kernel(x, adj, w_j, W1, W2):
    B, N, K, D = x.shape
    H = W1.shape[1]
    DOUT = W2.shape[1]

    adj_i8 = adj.view(jnp.int8)
    w1b = W1.astype(jnp.bfloat16)
    w2b = W2.astype(jnp.bfloat16)

    return pl.pallas_call(
        _gnn_kernel,
        grid=(B,),
        in_specs=[
            pl.BlockSpec(memory_space=pltpu.MemorySpace.HBM),
            pl.BlockSpec((1, K, N, N), lambda b: (b, 0, 0, 0)),
            pl.BlockSpec((1, N, K), lambda b: (b, 0, 0)),
            pl.BlockSpec((D, H), lambda b: (0, 0)),
            pl.BlockSpec((H, DOUT), lambda b: (0, 0)),
        ],
        out_specs=pl.BlockSpec((1, N, DOUT), lambda b: (b, 0, 0)),
        out_shape=jax.ShapeDtypeStruct((B, N, DOUT), jnp.float32),
        scratch_shapes=[
            pltpu.VMEM((N, H), jnp.float32),
            pltpu.VMEM((2, K, N, D), jnp.float32),
            pltpu.SemaphoreType.DMA((2, K)),
        ],
    )(x, adj_i8, w_j, w1b, w2b)


# speedup vs baseline: 2.6726x; 1.2464x over previous
import jax
import jax.numpy as jnp
from jax.experimental import pallas as pl
from jax.experimental.pallas import tpu as pltpu


def _gnn_kernel(x_hbm, adj_ref, wj_ref, w1_ref, w2_ref, out_ref,
                acc_ref, xs_ref, y_ref, sem):
    b = pl.program_id(0)
    nb = pl.num_programs(0)
    nk = adj_ref.shape[1]
    slot = jax.lax.rem(b, 2)
    nxt = jax.lax.rem(b + 1, 2)

    def _issue(bb, ss):
        for k in range(nk):
            pltpu.make_async_copy(
                x_hbm.at[bb, :, k, :], xs_ref.at[ss, k], sem.at[ss, k]
            ).start()

    @pl.when(b == 0)
    def _():
        _issue(0, 0)

    @pl.when(b + 1 < nb)
    def _():
        _issue(b + 1, nxt)

    wj = wj_ref[0]

    for k in range(nk):
        pltpu.make_async_copy(
            x_hbm.at[b, :, k, :], xs_ref.at[slot, k], sem.at[slot, k]
        ).wait()
        xs = xs_ref[slot, k].astype(jnp.bfloat16)
        y_ref[k] = jnp.dot(
            xs, w1_ref[...], preferred_element_type=jnp.float32
        ).astype(jnp.bfloat16)

    for k in range(nk):
        a = adj_ref[0, k].astype(jnp.bfloat16)
        m = jnp.dot(a, y_ref[k], preferred_element_type=jnp.float32)
        phi = jnp.maximum(m, 0.0) * wj[:, k:k + 1].astype(jnp.float32)
        if k < 2:
            acc_ref[k] = phi
        else:
            acc_ref[k % 2] = acc_ref[k % 2] + phi

    s = (acc_ref[0] + acc_ref[1]).astype(jnp.bfloat16)
    out_ref[0] = jnp.dot(s, w2_ref[...], preferred_element_type=jnp.float32)


def kernel(x, adj, w_j, W1, W2):
    B, N, K, D = x.shape
    H = W1.shape[1]
    DOUT = W2.shape[1]

    adj_i8 = adj.view(jnp.int8)
    w1b = W1.astype(jnp.bfloat16)
    w2b = W2.astype(jnp.bfloat16)

    return pl.pallas_call(
        _gnn_kernel,
        grid=(B,),
        in_specs=[
            pl.BlockSpec(memory_space=pltpu.MemorySpace.HBM),
            pl.BlockSpec((1, K, N, N), lambda b: (b, 0, 0, 0)),
            pl.BlockSpec((1, N, K), lambda b: (b, 0, 0)),
            pl.BlockSpec((D, H), lambda b: (0, 0)),
            pl.BlockSpec((H, DOUT), lambda b: (0, 0)),
        ],
        out_specs=pl.BlockSpec((1, N, DOUT), lambda b: (b, 0, 0)),
        out_shape=jax.ShapeDtypeStruct((B, N, DOUT), jnp.float32),
        scratch_shapes=[
            pltpu.VMEM((2, N, H), jnp.float32),
            pltpu.VMEM((2, K, N, D), jnp.float32),
            pltpu.VMEM((K, N, H), jnp.bfloat16),
            pltpu.SemaphoreType.DMA((2, K)),
        ],
    )(x, adj_i8, w_j, w1b, w2b)
